# baseline (speedup 1.0000x reference)
import functools

import jax
import jax.numpy as jnp
from jax import lax
from jax.experimental import pallas as pl
from jax.experimental.pallas import tpu as pltpu
from jax.experimental.pallas import tpu_sc as plsc

_N, _E, _D, _H, _OUT, _NG = 10000, 320000, 128, 128, 64, 64
_NC, _NS = 2, 16
_NW = _NC * _NS
_NP = 10240
_RPS = _NP // _NS
_CB = 128
_CPW = 80
_GC = 40
_EPW = _CB * _CPW
_EPAD = _EPW * _NW

_mesh = plsc.VectorSubcoreMesh(core_axis_name="c", subcore_axis_name="s")


@functools.partial(
    pl.kernel,
    mesh=_mesh,
    out_type=jax.ShapeDtypeStruct((_NC, _NP, 16), jnp.float32),
    scratch_types=[
        pltpu.VMEM((_CPW, _CB), jnp.int32),
        pltpu.VMEM((_CB, 16), jnp.float32),
        pltpu.VMEM((_CB, 16), jnp.float32),
        pltpu.VMEM_SHARED((_NP, 16), jnp.float32),
    ],
)
def _sc_degree(dst_hbm, out_hbm, idx_v, ones_v, zero_v, acc_sh):
    c = lax.axis_index("c")
    s = lax.axis_index("s")
    wid = s * _NC + c

    def _fill(i, carry):
        ones_v[i] = jnp.full((16,), 1.0, jnp.float32)
        zero_v[i] = jnp.zeros((16,), jnp.float32)
        return carry

    lax.fori_loop(0, _CB, _fill, 0)
    for q in range(_RPS // _CB):
        pltpu.sync_copy(zero_v, acc_sh.at[pl.ds(s * _RPS + q * _CB, _CB)])
    plsc.subcore_barrier()

    pltpu.sync_copy(dst_hbm.at[wid], idx_v)

    def _body(k, carry):
        pltpu.sync_copy(ones_v, acc_sh.at[idx_v.at[k]], add=True)
        return carry

    lax.fori_loop(0, _CPW, _body, 0)
    plsc.subcore_barrier()
    for q in range(_RPS // _CB):
        r = s * _RPS + q * _CB
        pltpu.sync_copy(acc_sh.at[pl.ds(r, _CB)], zero_v)
        pltpu.sync_copy(zero_v, out_hbm.at[c].at[pl.ds(r, _CB)])


@functools.partial(
    pl.kernel,
    mesh=_mesh,
    out_type=jax.ShapeDtypeStruct((_NC, _NP, _D), jnp.float32),
    scratch_types=[
        pltpu.VMEM((_GC, _CB), jnp.int32),
        pltpu.VMEM((_GC, _CB), jnp.int32),
        pltpu.VMEM((_CB, _D), jnp.float32),
        pltpu.VMEM((_CB, _D), jnp.float32),
        pltpu.VMEM_SHARED((_NP, _D), jnp.float32),
        pltpu.SemaphoreType.DMA,
        pltpu.SemaphoreType.DMA,
    ],
)
def _sc_scatter(u_hbm, src_hbm, dst_hbm, out_hbm, idxs_v, idxd_v, rows_a,
                rows_b, acc_sh, sem_a, sem_b):
    c = lax.axis_index("c")
    s = lax.axis_index("s")
    wid = s * _NC + c

    def _zero(i, carry):
        for j in range(_D // 16):
            rows_a[i, pl.ds(j * 16, 16)] = jnp.zeros((16,), jnp.float32)
        return carry

    lax.fori_loop(0, _CB, _zero, 0)
    for q in range(_RPS // _CB):
        pltpu.sync_copy(rows_a, acc_sh.at[pl.ds(s * _RPS + q * _CB, _CB)])
    plsc.subcore_barrier()

    for g in range(_CPW // _GC):
        pltpu.sync_copy(src_hbm.at[wid].at[pl.ds(g * _GC, _GC)], idxs_v)
        pltpu.sync_copy(dst_hbm.at[wid].at[pl.ds(g * _GC, _GC)], idxd_v)

        pltpu.async_copy(u_hbm.at[idxs_v.at[0]], rows_a, sem_a)
        pltpu.async_copy(u_hbm.at[idxs_v.at[1]], rows_b, sem_b)

        def _body(p, carry):
            k = p * 2
            pltpu.make_async_copy(u_hbm.at[idxs_v.at[k]], rows_a, sem_a).wait()
            pltpu.sync_copy(rows_a, acc_sh.at[idxd_v.at[k]], add=True)

            @pl.when(p < _GC // 2 - 1)
            def _():
                pltpu.async_copy(u_hbm.at[idxs_v.at[k + 2]], rows_a, sem_a)

            pltpu.make_async_copy(
                u_hbm.at[idxs_v.at[k + 1]], rows_b, sem_b).wait()
            pltpu.sync_copy(rows_b, acc_sh.at[idxd_v.at[k + 1]], add=True)

            @pl.when(p < _GC // 2 - 1)
            def _():
                pltpu.async_copy(u_hbm.at[idxs_v.at[k + 3]], rows_b, sem_b)

            return carry

        lax.fori_loop(0, _GC // 2, _body, 0)
    plsc.subcore_barrier()
    for q in range(_RPS // _CB):
        r = s * _RPS + q * _CB
        pltpu.sync_copy(acc_sh.at[pl.ds(r, _CB)], rows_a)
        pltpu.sync_copy(rows_a, out_hbm.at[c].at[pl.ds(r, _CB)])


def _tc_enc(deg_ref, x_ref, we_ref, be_ref, w0_ref, dis_ref, u_ref):
    deg = deg_ref[0][:, 0:1] + deg_ref[1][:, 0:1] + 1.0
    rows = lax.broadcasted_iota(jnp.int32, (_NP, 1), 0)
    dis = jnp.where(rows < _N, lax.rsqrt(deg), 0.0)
    dis_ref[...] = dis
    h = jnp.dot(x_ref[...], we_ref[...],
                preferred_element_type=jnp.float32) + be_ref[...]
    u_ref[...] = jnp.dot(h, w0_ref[...],
                         preferred_element_type=jnp.float32) * dis


def _tc_bridge0(s2_ref, u_ref, dis_ref, bc_ref, wn_ref, un_ref):
    dis = dis_ref[...]
    h = dis * (s2_ref[0] + s2_ref[1] + u_ref[...]) + bc_ref[...]
    un_ref[...] = jnp.dot(h, wn_ref[...],
                          preferred_element_type=jnp.float32) * dis


def _tc_bridge(s2_ref, u_ref, dis_ref, bc_ref, g_ref, bb_ref, wn_ref, un_ref):
    dis = dis_ref[...]
    pre = dis * (s2_ref[0] + s2_ref[1] + u_ref[...]) + bc_ref[...]
    mu = jnp.mean(pre, axis=-1, keepdims=True)
    var = jnp.mean((pre - mu) ** 2, axis=-1, keepdims=True)
    h = jnp.maximum(
        (pre - mu) * lax.rsqrt(var + 1e-5) * g_ref[...] + bb_ref[...], 0.0)
    un_ref[...] = jnp.dot(h, wn_ref[...],
                          preferred_element_type=jnp.float32) * dis


def _tc_final(s2_ref, u_ref, dis_ref, bc_ref, g_ref, bb_ref, batch_ref,
              w1_ref, b1_ref, w2_ref, b2_ref, out_ref):
    dis = dis_ref[...]
    pre = dis * (s2_ref[0] + s2_ref[1] + u_ref[...]) + bc_ref[...]
    mu = jnp.mean(pre, axis=-1, keepdims=True)
    var = jnp.mean((pre - mu) ** 2, axis=-1, keepdims=True)
    h = jnp.maximum(
        (pre - mu) * lax.rsqrt(var + 1e-5) * g_ref[...] + bb_ref[...], 0.0)
    gids = lax.broadcasted_iota(jnp.int32, (_NG, _NP), 0)
    mask = jnp.where(gids == batch_ref[...], 1.0, 0.0)
    y = jnp.dot(mask, h, preferred_element_type=jnp.float32)
    t = jnp.maximum(
        jnp.dot(y, w1_ref[...], preferred_element_type=jnp.float32)
        + b1_ref[...], 0.0)
    out_ref[...] = (jnp.dot(t, w2_ref[...], preferred_element_type=jnp.float32)
                    + b2_ref[...])


_f32 = jnp.float32
_enc_call = pl.pallas_call(
    _tc_enc,
    out_shape=(jax.ShapeDtypeStruct((_NP, 1), _f32),
               jax.ShapeDtypeStruct((_NP, _H), _f32)))
_bridge0_call = pl.pallas_call(
    _tc_bridge0, out_shape=jax.ShapeDtypeStruct((_NP, _H), _f32))
_bridge_call = pl.pallas_call(
    _tc_bridge, out_shape=jax.ShapeDtypeStruct((_NP, _H), _f32))
_final_call = pl.pallas_call(
    _tc_final, out_shape=jax.ShapeDtypeStruct((_NG, _OUT), _f32))


def kernel(x, edge_index, batch_idx, W_enc, b_enc, W_conv0, b_conv0,
           W_conv1, b_conv1, W_conv2, b_conv2, ln_g0, ln_b0, ln_g1, ln_b1,
           ln_g2, ln_b2, W1, b1, W2, b2):
    pad_e = jnp.full((_EPAD - _E,), _N, jnp.int32)
    srcp = jnp.concatenate([edge_index[0], pad_e]).reshape(_NW, _CPW, _CB)
    dstp = jnp.concatenate([edge_index[1], pad_e]).reshape(_NW, _CPW, _CB)
    xp = jnp.pad(x, ((0, _NP - _N), (0, 0)))
    bip = jnp.concatenate(
        [batch_idx, jnp.full((_NP - _N,), _NG, jnp.int32)]).reshape(1, _NP)
    be = b_enc.reshape(1, _H)
    bc0, bc1, bc2 = (b_conv0.reshape(1, _H), b_conv1.reshape(1, _H),
                     b_conv2.reshape(1, _H))
    g0, g1, g2 = ln_g0.reshape(1, _H), ln_g1.reshape(1, _H), ln_g2.reshape(1, _H)
    lb0, lb1, lb2 = (ln_b0.reshape(1, _H), ln_b1.reshape(1, _H),
                     ln_b2.reshape(1, _H))

    deg2 = _sc_degree(dstp)
    dis, u0 = _enc_call(deg2, xp, W_enc, be, W_conv0)
    s0 = _sc_scatter(u0, srcp, dstp)
    u1 = _bridge0_call(s0, u0, dis, bc0, W_conv0)
    s1 = _sc_scatter(u1, srcp, dstp)
    u2 = _bridge_call(s1, u1, dis, bc0, g0, lb0, W_conv1)
    s2 = _sc_scatter(u2, srcp, dstp)
    u3 = _bridge_call(s2, u2, dis, bc1, g1, lb1, W_conv2)
    s3 = _sc_scatter(u3, srcp, dstp)
    out = _final_call(s3, u3, dis, bc2, g2, lb2, bip,
                      W1, b1.reshape(1, _H), W2, b2.reshape(1, _OUT))
    return out


# speedup vs baseline: 1.0922x; 1.0922x over previous
import functools

import jax
import jax.numpy as jnp
from jax import lax
from jax.experimental import pallas as pl
from jax.experimental.pallas import tpu as pltpu
from jax.experimental.pallas import tpu_sc as plsc

_N, _E, _D, _H, _OUT, _NG = 10000, 320000, 128, 128, 64, 64
_NC, _NS = 2, 16
_NW = _NC * _NS
_NP = 10240
_RPS = _NP // _NS
_CB = 128
_NCH = 2560
_GC = 8
_GD = 2
_NBUF = 2
_EPAD = _NCH * _CB
_QA = 128
_QB = 32
_QDEG = _NCH // _NW

_mesh = plsc.VectorSubcoreMesh(core_axis_name="c", subcore_axis_name="s")


@functools.partial(
    pl.kernel,
    mesh=_mesh,
    out_type=jax.ShapeDtypeStruct((_NC, _NP, 16), jnp.float32),
    scratch_types=[
        pltpu.VMEM((_GD, _CB), jnp.int32),
        pltpu.VMEM((_CB, 16), jnp.float32),
        pltpu.VMEM((64, 16), jnp.float32),
        pltpu.VMEM_SHARED((_NP, 16), jnp.float32),
    ],
)
def _sc_degree(dst_hbm, out_hbm, idx_v, ones_v, zero_v, acc_sh):
    c = lax.axis_index("c")
    s = lax.axis_index("s")
    wid = s * _NC + c

    def _fill(i, carry):
        ones_v[i] = jnp.full((16,), 1.0, jnp.float32)
        return carry

    def _fill0(i, carry):
        zero_v[i] = jnp.zeros((16,), jnp.float32)
        return carry

    lax.fori_loop(0, _CB, _fill, 0)
    lax.fori_loop(0, 64, _fill0, 0)
    for q in range(_RPS // 64):
        pltpu.sync_copy(zero_v, acc_sh.at[pl.ds(s * _RPS + q * 64, 64)])
    plsc.subcore_barrier()

    cbase = wid * _QDEG

    def _group(g, carry):
        pltpu.sync_copy(dst_hbm.at[pl.ds(cbase + g * _GD, _GD)], idx_v)

        def _body(k, carry2):
            pltpu.sync_copy(ones_v, acc_sh.at[idx_v.at[k]], add=True)
            return carry2

        lax.fori_loop(0, _GD, _body, 0)
        return carry

    lax.fori_loop(0, _QDEG // _GD, _group, 0)
    plsc.subcore_barrier()
    for q in range(_RPS // 64):
        r = s * _RPS + q * 64
        pltpu.sync_copy(acc_sh.at[pl.ds(r, 64)], zero_v)
        pltpu.sync_copy(zero_v, out_hbm.at[c].at[pl.ds(r, 64)])


@functools.partial(
    pl.kernel,
    mesh=_mesh,
    out_type=jax.ShapeDtypeStruct((_NC, _NP, _D), jnp.float32),
    scratch_types=[
        pltpu.VMEM((_GC, _CB), jnp.int32),
        pltpu.VMEM((_GC, _CB), jnp.int32),
        pltpu.VMEM((_NBUF, _CB, _D), jnp.float32),
        pltpu.VMEM_SHARED((_NP, _D), jnp.float32),
        pltpu.SemaphoreType.DMA,
        pltpu.SemaphoreType.DMA,
    ],
)
def _sc_scatter(u_hbm, src_hbm, dst_hbm, out_hbm, idxs_v, idxd_v, rows_v,
                acc_sh, sem0, sem1):
    c = lax.axis_index("c")
    s = lax.axis_index("s")
    sems = (sem0, sem1)

    def _zero(i, carry):
        for j in range(_D // 16):
            rows_v[0, i, pl.ds(j * 16, 16)] = jnp.zeros((16,), jnp.float32)
        return carry

    lax.fori_loop(0, _CB, _zero, 0)
    for q in range(_RPS // _CB):
        pltpu.sync_copy(rows_v.at[0],
                        acc_sh.at[pl.ds(s * _RPS + q * _CB, _CB)])
    plsc.subcore_barrier()

    cbase = lax.select(c == 0, s * _QA, 16 * _QA + s * _QB)
    ngr = lax.select(c == 0, _QA // _GC, _QB // _GC)

    for g in range(_QA // _GC):

        @pl.when(g < ngr)
        def _grp():
            gbase = cbase + g * _GC
            pltpu.sync_copy(src_hbm.at[pl.ds(gbase, _GC)], idxs_v)
            pltpu.sync_copy(dst_hbm.at[pl.ds(gbase, _GC)], idxd_v)

            for b in range(_NBUF):
                pltpu.async_copy(u_hbm.at[idxs_v.at[b]], rows_v.at[b],
                                 sems[b])

            def _body(p, carry2):
                k = p * _NBUF
                for b in range(_NBUF):
                    pltpu.make_async_copy(
                        u_hbm.at[idxs_v.at[k + b]], rows_v.at[b],
                        sems[b]).wait()
                    pltpu.sync_copy(rows_v.at[b],
                                    acc_sh.at[idxd_v.at[k + b]], add=True)

                    @pl.when(k + b + _NBUF < _GC)
                    def _():
                        pltpu.async_copy(u_hbm.at[idxs_v.at[k + b + _NBUF]],
                                         rows_v.at[b], sems[b])

                return carry2

            lax.fori_loop(0, _GC // _NBUF, _body, 0)

    plsc.subcore_barrier()
    for q in range(_RPS // _CB):
        r = s * _RPS + q * _CB
        pltpu.sync_copy(acc_sh.at[pl.ds(r, _CB)], rows_v.at[0])
        pltpu.sync_copy(rows_v.at[0], out_hbm.at[c].at[pl.ds(r, _CB)])


def _tc_enc(deg_ref, x_ref, we_ref, be_ref, w0_ref, dis_ref, u_ref):
    deg = deg_ref[0][:, 0:1] + deg_ref[1][:, 0:1] + 1.0
    rows = lax.broadcasted_iota(jnp.int32, (_NP, 1), 0)
    dis = jnp.where(rows < _N, lax.rsqrt(deg), 0.0)
    dis_ref[...] = dis
    h = jnp.dot(x_ref[...], we_ref[...],
                preferred_element_type=jnp.float32) + be_ref[...]
    u_ref[...] = jnp.dot(h, w0_ref[...],
                         preferred_element_type=jnp.float32) * dis


def _tc_bridge0(s2_ref, u_ref, dis_ref, bc_ref, wn_ref, un_ref):
    dis = dis_ref[...]
    h = dis * (s2_ref[0] + s2_ref[1] + u_ref[...]) + bc_ref[...]
    un_ref[...] = jnp.dot(h, wn_ref[...],
                          preferred_element_type=jnp.float32) * dis


def _tc_bridge(s2_ref, u_ref, dis_ref, bc_ref, g_ref, bb_ref, wn_ref, un_ref):
    dis = dis_ref[...]
    pre = dis * (s2_ref[0] + s2_ref[1] + u_ref[...]) + bc_ref[...]
    mu = jnp.mean(pre, axis=-1, keepdims=True)
    var = jnp.mean((pre - mu) ** 2, axis=-1, keepdims=True)
    h = jnp.maximum(
        (pre - mu) * lax.rsqrt(var + 1e-5) * g_ref[...] + bb_ref[...], 0.0)
    un_ref[...] = jnp.dot(h, wn_ref[...],
                          preferred_element_type=jnp.float32) * dis


def _tc_final(s2_ref, u_ref, dis_ref, bc_ref, g_ref, bb_ref, batch_ref,
              w1_ref, b1_ref, w2_ref, b2_ref, out_ref):
    dis = dis_ref[...]
    pre = dis * (s2_ref[0] + s2_ref[1] + u_ref[...]) + bc_ref[...]
    mu = jnp.mean(pre, axis=-1, keepdims=True)
    var = jnp.mean((pre - mu) ** 2, axis=-1, keepdims=True)
    h = jnp.maximum(
        (pre - mu) * lax.rsqrt(var + 1e-5) * g_ref[...] + bb_ref[...], 0.0)
    gids = lax.broadcasted_iota(jnp.int32, (_NG, _NP), 0)
    mask = jnp.where(gids == batch_ref[...], 1.0, 0.0)
    y = jnp.dot(mask, h, preferred_element_type=jnp.float32)
    t = jnp.maximum(
        jnp.dot(y, w1_ref[...], preferred_element_type=jnp.float32)
        + b1_ref[...], 0.0)
    out_ref[...] = (jnp.dot(t, w2_ref[...], preferred_element_type=jnp.float32)
                    + b2_ref[...])


_f32 = jnp.float32
_enc_call = pl.pallas_call(
    _tc_enc,
    out_shape=(jax.ShapeDtypeStruct((_NP, 1), _f32),
               jax.ShapeDtypeStruct((_NP, _H), _f32)))
_bridge0_call = pl.pallas_call(
    _tc_bridge0, out_shape=jax.ShapeDtypeStruct((_NP, _H), _f32))
_bridge_call = pl.pallas_call(
    _tc_bridge, out_shape=jax.ShapeDtypeStruct((_NP, _H), _f32))
_final_call = pl.pallas_call(
    _tc_final, out_shape=jax.ShapeDtypeStruct((_NG, _OUT), _f32))


def kernel(x, edge_index, batch_idx, W_enc, b_enc, W_conv0, b_conv0,
           W_conv1, b_conv1, W_conv2, b_conv2, ln_g0, ln_b0, ln_g1, ln_b1,
           ln_g2, ln_b2, W1, b1, W2, b2):
    pad_e = jnp.full((_EPAD - _E,), _N, jnp.int32)
    srcp = jnp.concatenate([edge_index[0], pad_e]).reshape(_NCH, _CB)
    dstp = jnp.concatenate([edge_index[1], pad_e]).reshape(_NCH, _CB)
    xp = jnp.pad(x, ((0, _NP - _N), (0, 0)))
    bip = jnp.concatenate(
        [batch_idx, jnp.full((_NP - _N,), _NG, jnp.int32)]).reshape(1, _NP)
    be = b_enc.reshape(1, _H)
    bc0, bc1, bc2 = (b_conv0.reshape(1, _H), b_conv1.reshape(1, _H),
                     b_conv2.reshape(1, _H))
    g0, g1, g2 = ln_g0.reshape(1, _H), ln_g1.reshape(1, _H), ln_g2.reshape(1, _H)
    lb0, lb1, lb2 = (ln_b0.reshape(1, _H), ln_b1.reshape(1, _H),
                     ln_b2.reshape(1, _H))

    deg2 = _sc_degree(dstp)
    dis, u0 = _enc_call(deg2, xp, W_enc, be, W_conv0)
    s0 = _sc_scatter(u0, srcp, dstp)
    u1 = _bridge0_call(s0, u0, dis, bc0, W_conv0)
    s1 = _sc_scatter(u1, srcp, dstp)
    u2 = _bridge_call(s1, u1, dis, bc0, g0, lb0, W_conv1)
    s2 = _sc_scatter(u2, srcp, dstp)
    u3 = _bridge_call(s2, u2, dis, bc1, g1, lb1, W_conv2)
    s3 = _sc_scatter(u3, srcp, dstp)
    out = _final_call(s3, u3, dis, bc2, g2, lb2, bip,
                      W1, b1.reshape(1, _H), W2, b2.reshape(1, _OUT))
    return out
